# Initial kernel scaffold; baseline (speedup 1.0000x reference)
import jax
import jax.numpy as jnp
from jax.experimental import pallas as pl


def kernel(x, Wr, We, be):
    raise NotImplementedError("write your pallas kernel here")


# speedup vs baseline: 3.2976x; 3.2976x over previous
import functools

import jax
import jax.numpy as jnp
from jax.experimental import pallas as pl
from jax.experimental.pallas import tpu as pltpu

E = 8
TOP_K = 2
D = 768
BLK = 512

_NEG_INF = -1e30


def _router_weights(logits):
    m1 = jnp.max(logits, axis=-1)
    i1 = jnp.argmax(logits, axis=-1)
    cols = jax.lax.broadcasted_iota(jnp.int32, logits.shape, 1)
    masked = jnp.where(cols == i1[:, None], _NEG_INF, logits)
    m2 = jnp.max(masked, axis=-1)
    i2 = jnp.argmax(masked, axis=-1)
    w1 = 1.0 / (1.0 + jnp.exp(m2 - m1))
    w2 = 1.0 - w1
    return i1, i2, w1, w2


def _moe_body(x_ref, wr_ref, we_ref, be_ref, out_ref, w_scr, i1_scr, i2_scr):
    e = pl.program_id(1)

    @pl.when(e == 0)
    def _():
        logits = jnp.dot(x_ref[...], wr_ref[...])
        i1, i2, w1, w2 = _router_weights(logits)
        w_scr[:, 0] = w1
        w_scr[:, 1] = w2
        i1_scr[:, 0] = i1
        i2_scr[:, 0] = i2

    h = jnp.dot(x_ref[...], we_ref[0]) + be_ref[0]
    h = 0.5 * h * (1.0 + jax.lax.erf(h * 0.7071067811865476))
    i1 = i1_scr[:, 0]
    i2 = i2_scr[:, 0]
    w_e = jnp.where(i1 == e, w_scr[:, 0], 0.0) + jnp.where(i2 == e, w_scr[:, 1], 0.0)
    contrib = w_e[:, None] * h

    @pl.when(e == 0)
    def _():
        out_ref[...] = contrib

    @pl.when(e != 0)
    def _():
        out_ref[...] += contrib


@jax.jit
def kernel(x, Wr, We, be):
    Bx, Sx, Dx = x.shape
    T = Bx * Sx
    xf = x.reshape(T, Dx)
    nb = T // BLK

    out = pl.pallas_call(
        _moe_body,
        grid=(nb, E),
        in_specs=[
            pl.BlockSpec((BLK, Dx), lambda b, e: (b, 0)),
            pl.BlockSpec((Dx, E), lambda b, e: (0, 0)),
            pl.BlockSpec((1, Dx, Dx), lambda b, e: (e, 0, 0)),
            pl.BlockSpec((1, 1, Dx), lambda b, e: (e, 0, 0)),
        ],
        out_specs=pl.BlockSpec((BLK, Dx), lambda b, e: (b, 0)),
        out_shape=jax.ShapeDtypeStruct((T, Dx), jnp.float32),
        scratch_shapes=[
            pltpu.VMEM((BLK, 2), jnp.float32),
            pltpu.VMEM((BLK, 1), jnp.int32),
            pltpu.VMEM((BLK, 1), jnp.int32),
        ],
        compiler_params=pltpu.CompilerParams(
            dimension_semantics=("parallel", "arbitrary"),
        ),
    )(xf, Wr, We, be.reshape(E, 1, Dx))
    return out.reshape(Bx, Sx, Dx)
